# baseline (speedup 1.0000x reference)
import functools

import jax
import jax.numpy as jnp
from jax import lax
from jax.experimental import pallas as pl
from jax.experimental.pallas import tpu as pltpu
from jax.experimental.pallas import tpu_sc as plsc

N = 100000
HIDDEN = 256
NUM_EMB = 95

NC = 2
NS = 16
NW = NC * NS

CHUNK = 80
GROUPS = CHUNK // 16
NCHUNKS = N // CHUNK
NMAX = -(-NCHUNKS // NW)
NFULL = NCHUNKS - NW * (NMAX - 1)
NBUF = 2
NROUND = NMAX // NBUF
TBL = NUM_EMB * HIDDEN


def _swish_table(w):
    def body(w_ref, o_ref):
        v = w_ref[...]
        o_ref[...] = v * (1.0 / (1.0 + jnp.exp(-v)))
    return pl.pallas_call(
        body,
        out_shape=jax.ShapeDtypeStruct(w.shape, w.dtype),
    )(w)


def _make_sc_lookup():
    mesh = plsc.VectorSubcoreMesh(core_axis_name="c", subcore_axis_name="s")

    @functools.partial(
        pl.kernel,
        mesh=mesh,
        out_type=jax.ShapeDtypeStruct((N * HIDDEN,), jnp.float32),
        scratch_types=[
            pltpu.VMEM((NMAX * CHUNK + 16,), jnp.int32),
            pltpu.VMEM((TBL,), jnp.float32),
        ] + [pltpu.VMEM((CHUNK * HIDDEN,), jnp.float32)] * NBUF
          + [pltpu.VMEM_SHARED((NS, NBUF, CHUNK * HIDDEN), jnp.float32)]
          + [pltpu.SemaphoreType.DMA]
          + [pltpu.SemaphoreType.DMA] * NBUF,
        compiler_params=pltpu.CompilerParams(needs_layout_passes=False),
    )
    def sc_lookup(table_hbm, idx_hbm, out_hbm, idx_all, table_v, *rest):
        rows = rest[:NBUF]
        shared = rest[NBUF]
        isem = rest[NBUF + 1]
        wsems = rest[NBUF + 2:]
        sid = lax.axis_index("s")
        w = lax.axis_index("s") * NC + lax.axis_index("c")
        last = w < NFULL

        def idx_base(i):
            return pl.multiple_of((w + i * NW) * CHUNK, CHUNK)

        def out_base(i):
            return pl.multiple_of((w + i * NW) * CHUNK * HIDDEN, CHUNK * HIDDEN)

        icps = [
            pltpu.make_async_copy(
                idx_hbm.at[pl.ds(idx_base(i), CHUNK)],
                idx_all.at[pl.ds(i * CHUNK, CHUNK)],
                isem,
            )
            for i in range(NMAX)
        ]
        tcp = pltpu.make_async_copy(table_hbm, table_v, isem)
        tcp.start()
        for i in range(NMAX - 1):
            icps[i].start()
        pl.when(last)(icps[NMAX - 1].start)
        tcp.wait()
        for i in range(NMAX - 1):
            icps[i].wait()
        pl.when(last)(icps[NMAX - 1].wait)

        lane = lax.iota(jnp.int32, 16)

        def compute_chunk(i, b):
            @plsc.parallel_loop(0, CHUNK, 1, unroll=4)
            def row(r):
                s = idx_all[pl.ds(i * CHUNK + r, 16)][0]
                base = s * HIDDEN
                dst = r * HIDDEN
                for k in range(HIDDEN // 16):
                    rows[b][pl.ds(dst + k * 16, 16)] = table_v[pl.ds(base + k * 16, 16)]

        def wcp(i, b):
            return pltpu.make_async_copy(
                shared.at[sid, b],
                out_hbm.at[pl.ds(out_base(i), CHUNK * HIDDEN)],
                wsems[b],
            )

        def around(j, carry):
            glast = jnp.logical_or(j < NROUND - 1, last)
            for b in range(NBUF):
                i = NBUF * j + b
                guard = glast if b == NBUF - 1 else None

                def do(b=b, i=i, j=j):
                    compute_chunk(i, b)
                    pl.when(j > 0)(wcp(0, b).wait)
                    pltpu.sync_copy(rows[b], shared.at[sid, b])
                    wcp(i, b).start()

                if guard is None:
                    do()
                else:
                    pl.when(guard)(do)
            return carry

        lax.fori_loop(0, NROUND, around, 0)
        for b in range(NBUF - 1):
            wcp(0, b).wait()
        pl.when(last)(wcp(0, NBUF - 1).wait)

    return sc_lookup


_sc_lookup = _make_sc_lookup()


def kernel(x, emb_weight):
    act_table = _swish_table(emb_weight)
    flat = _sc_lookup(act_table.reshape(-1), x.astype(jnp.int32))
    return flat.reshape(N, HIDDEN)


# speedup vs baseline: 1.2106x; 1.2106x over previous
import functools

import jax
import jax.numpy as jnp
from jax import lax
from jax.experimental import pallas as pl
from jax.experimental.pallas import tpu as pltpu
from jax.experimental.pallas import tpu_sc as plsc

N = 100000
HIDDEN = 256
NUM_EMB = 95

NC = 2
NS = 16
NW = NC * NS

CHUNK = 160
GROUPS = CHUNK // 16
NCHUNKS = N // CHUNK
NMAX = -(-NCHUNKS // NW)
NFULL = NCHUNKS - NW * (NMAX - 1)
NBUF = 2
NROUND = NMAX // NBUF
TBL = NUM_EMB * HIDDEN


def _swish_table(w):
    def body(w_ref, o_ref):
        v = w_ref[...]
        o_ref[...] = v * (1.0 / (1.0 + jnp.exp(-v)))
    return pl.pallas_call(
        body,
        out_shape=jax.ShapeDtypeStruct(w.shape, w.dtype),
    )(w)


def _make_sc_lookup():
    mesh = plsc.VectorSubcoreMesh(core_axis_name="c", subcore_axis_name="s")

    @functools.partial(
        pl.kernel,
        mesh=mesh,
        out_type=jax.ShapeDtypeStruct((N * HIDDEN,), jnp.float32),
        scratch_types=[
            pltpu.VMEM((NMAX * CHUNK + 16,), jnp.int32),
            pltpu.VMEM((TBL,), jnp.float32),
        ] + [pltpu.VMEM((CHUNK * HIDDEN,), jnp.float32)] * NBUF
          + [pltpu.SemaphoreType.DMA]
          + [pltpu.SemaphoreType.DMA] * NBUF,
        compiler_params=pltpu.CompilerParams(needs_layout_passes=False),
    )
    def sc_lookup(table_hbm, idx_hbm, out_hbm, idx_all, table_v, *rest):
        rows = rest[:NBUF]
        isem = rest[NBUF]
        wsems = rest[NBUF + 1:]
        w = lax.axis_index("s") * NC + lax.axis_index("c")
        last = w < NFULL

        def idx_base(i):
            return pl.multiple_of((w + i * NW) * CHUNK, CHUNK)

        def out_base(i):
            return pl.multiple_of((w + i * NW) * CHUNK * HIDDEN, CHUNK * HIDDEN)

        icps = [
            pltpu.make_async_copy(
                idx_hbm.at[pl.ds(idx_base(i), CHUNK)],
                idx_all.at[pl.ds(i * CHUNK, CHUNK)],
                isem,
            )
            for i in range(NMAX)
        ]
        tcp = pltpu.make_async_copy(table_hbm, table_v, isem)
        tcp.start()
        for i in range(NMAX - 1):
            icps[i].start()
        pl.when(last)(icps[NMAX - 1].start)
        tcp.wait()
        for i in range(NMAX - 1):
            icps[i].wait()
        pl.when(last)(icps[NMAX - 1].wait)

        lane = lax.iota(jnp.int32, 16)

        def compute_chunk(i, b):
            @plsc.parallel_loop(0, CHUNK, 1, unroll=4)
            def row(r):
                s = idx_all[pl.ds(i * CHUNK + r, 16)][0]
                base = s * HIDDEN
                dst = r * HIDDEN
                for k in range(HIDDEN // 16):
                    rows[b][pl.ds(dst + k * 16, 16)] = table_v[pl.ds(base + k * 16, 16)]

        def wcp(i, b):
            return pltpu.make_async_copy(
                rows[b],
                out_hbm.at[pl.ds(out_base(i), CHUNK * HIDDEN)],
                wsems[b],
            )

        def around(j, carry):
            glast = jnp.logical_or(j < NROUND - 1, last)
            for b in range(NBUF):
                i = NBUF * j + b
                guard = glast if b == NBUF - 1 else None
                if b == NBUF - 1:
                    pl.when(j > 0)(wcp(0, b).wait)
                else:
                    pl.when(j > 0)(wcp(0, b).wait)

                def do(b=b, i=i):
                    compute_chunk(i, b)
                    wcp(i, b).start()

                if guard is None:
                    do()
                else:
                    pl.when(guard)(do)
            return carry

        lax.fori_loop(0, NROUND, around, 0)
        for b in range(NBUF - 1):
            wcp(0, b).wait()
        pl.when(last)(wcp(0, NBUF - 1).wait)

    return sc_lookup


_sc_lookup = _make_sc_lookup()


def kernel(x, emb_weight):
    act_table = _swish_table(emb_weight)
    flat = _sc_lookup(act_table.reshape(-1), x.astype(jnp.int32))
    return flat.reshape(N, HIDDEN)
